# baseline (speedup 1.0000x reference)
import functools

import jax
import jax.numpy as jnp
from jax import lax
from jax.experimental import pallas as pl
from jax.experimental.pallas import tpu as pltpu
from jax.experimental.pallas import tpu_sc as plsc

B, N, CDIM = 8, 2048, 64
S, K = 512, 16
ST, TS = 4, 128
NW = 32
ROWS = B * S * K
R_PER_W = ROWS // NW
CHUNK = 512


def _fps_body(p0_ref, p1_ref, p2_ref, c0_ref, c1_ref, c2_ref):
    p0 = p0_ref[...]
    p1 = p1_ref[...]
    p2 = p2_ref[...]
    lane = lax.broadcasted_iota(jnp.int32, (B, N), 1)
    lane_s = lax.broadcasted_iota(jnp.int32, (B, S), 1)

    def step(i, carry):
        dist, far, a0, a1, a2 = carry
        m = lane == far
        c0 = jnp.sum(jnp.where(m, p0, 0.0), axis=1, keepdims=True)
        c1 = jnp.sum(jnp.where(m, p1, 0.0), axis=1, keepdims=True)
        c2 = jnp.sum(jnp.where(m, p2, 0.0), axis=1, keepdims=True)
        sel = lane_s == i
        a0 = jnp.where(sel, c0, a0)
        a1 = jnp.where(sel, c1, a1)
        a2 = jnp.where(sel, c2, a2)
        d = ((p0 - c0) ** 2 + (p1 - c1) ** 2) + (p2 - c2) ** 2
        dist = jnp.minimum(dist, d)
        mx = jnp.max(dist, axis=1, keepdims=True)
        far = jnp.min(jnp.where(dist == mx, lane, N), axis=1, keepdims=True)
        return dist, far, a0, a1, a2

    dist0 = jnp.full((B, N), 1e10, dtype=jnp.float32)
    far0 = jnp.zeros((B, 1), dtype=jnp.int32)
    z = jnp.zeros((B, S), dtype=jnp.float32)
    init = (dist0, far0, z, z, z)
    _, _, a0, a1, a2 = lax.fori_loop(0, S, step, init)
    c0_ref[...] = a0
    c1_ref[...] = a1
    c2_ref[...] = a2


def _fps(p0, p1, p2, interpret=False):
    return pl.pallas_call(
        _fps_body,
        out_shape=[jax.ShapeDtypeStruct((B, S), jnp.float32)] * 3,
        interpret=interpret,
    )(p0, p1, p2)


def _knn_body(p0_ref, p1_ref, p2_ref, c0_ref, c1_ref, c2_ref, knn_ref):
    b = pl.program_id(0)
    p0 = p0_ref[...].reshape(1, N)
    p1 = p1_ref[...].reshape(1, N)
    p2 = p2_ref[...].reshape(1, N)

    lane = lax.broadcasted_iota(jnp.int32, (TS, N), 1)
    kidx = lax.broadcasted_iota(jnp.int32, (TS, K), 1)
    boff = b * N

    Ds = []
    for t in range(ST):
        c0 = c0_ref[0, t].reshape(TS, 1)
        c1 = c1_ref[0, t].reshape(TS, 1)
        c2 = c2_ref[0, t].reshape(TS, 1)
        Ds.append(((c0 - p0) ** 2 + (c1 - p1) ** 2) + (c2 - p2) ** 2)

    inf = jnp.float32(jnp.inf)

    def step(k, carry):
        dws = list(carry[0])
        iws = list(carry[1])
        accs = list(carry[2])
        for t in range(ST):
            dw, iw = dws[t], iws[t]
            elig = (Ds[t] > dw) | ((Ds[t] == dw) & (lane > iw))
            dv = jnp.where(elig, Ds[t], inf)
            iv = lane
            w = N
            while w > 128:
                h = w // 2
                take = dv[:, h:w] < dv[:, :h]
                dv = jnp.where(take, dv[:, h:w], dv[:, :h])
                iv = jnp.where(take, iv[:, h:w], iv[:, :h])
                w = h
            mn = jnp.min(dv, axis=1, keepdims=True)
            idx = jnp.min(jnp.where(dv == mn, iv, N), axis=1, keepdims=True)
            dws[t] = mn
            iws[t] = idx
            accs[t] = jnp.where(kidx == k, idx + boff, accs[t])
        return tuple(dws), tuple(iws), tuple(accs)

    acc0 = jnp.zeros((TS, K), jnp.int32)
    neg = jnp.full((TS, 1), -1.0, jnp.float32)
    zi = jnp.zeros((TS, 1), jnp.int32)
    res = lax.fori_loop(
        0, K, step, ((neg,) * ST, (zi,) * ST, (acc0,) * ST))
    for t in range(ST):
        knn_ref[0, t] = res[2][t]


def _knn(p0, p1, p2, c0r, c1r, c2r, interpret=False):
    pspec = pl.BlockSpec((1, 1, N), lambda b: (b, 0, 0))
    cspec = pl.BlockSpec((1, ST, TS, 1), lambda b: (b, 0, 0, 0))
    return pl.pallas_call(
        _knn_body,
        grid=(B,),
        in_specs=[pspec, pspec, pspec, cspec, cspec, cspec],
        out_specs=pl.BlockSpec((1, ST, TS, K), lambda b: (b, 0, 0, 0)),
        out_shape=jax.ShapeDtypeStruct((B, ST, TS, K), jnp.int32),
        interpret=interpret,
    )(p0.reshape(B, 1, N), p1.reshape(B, 1, N), p2.reshape(B, 1, N),
      c0r, c1r, c2r)


def _make_gather():
    mesh = plsc.VectorSubcoreMesh(
        core_axis_name="c", subcore_axis_name="s", num_cores=2, num_subcores=16
    )

    @functools.partial(
        pl.kernel,
        out_type=jax.ShapeDtypeStruct((ROWS, CDIM), jnp.float32),
        mesh=mesh,
        compiler_params=pltpu.CompilerParams(use_tc_tiling_on_sc=False),
        scratch_types=[
            pltpu.VMEM((CHUNK,), jnp.int32),
            pltpu.VMEM((CHUNK, CDIM), jnp.float32),
            pltpu.SemaphoreType.DMA,
        ],
    )
    def gather_rows(idx_hbm, x_hbm, out_hbm, idx_v, rows_v, sem):
        wid = lax.axis_index("s") * 2 + lax.axis_index("c")
        base = wid * R_PER_W
        for c in range(R_PER_W // CHUNK):
            off = base + c * CHUNK
            pltpu.sync_copy(idx_hbm.at[pl.ds(off, CHUNK)], idx_v)
            pltpu.async_copy(x_hbm.at[idx_v], rows_v, sem).wait()
            pltpu.sync_copy(rows_v, out_hbm.at[pl.ds(off, CHUNK)])

    return gather_rows


_gather_cache = []


def _get_gather():
    if not _gather_cache:
        _gather_cache.append(_make_gather())
    return _gather_cache[0]


def kernel(x):
    p0 = x[:, :, 0]
    p1 = x[:, :, 1]
    p2 = x[:, :, 2]
    c0, c1, c2 = _fps(p0, p1, p2)
    c0r = c0.reshape(B, ST, TS, 1)
    c1r = c1.reshape(B, ST, TS, 1)
    c2r = c2.reshape(B, ST, TS, 1)
    knn = _knn(p0, p1, p2, c0r, c1r, c2r)
    idx_flat = knn.reshape(ROWS)
    rows = _get_gather()(idx_flat, x.reshape(B * N, CDIM))
    return rows.reshape(B, S, K, CDIM)


# speedup vs baseline: 1.0550x; 1.0550x over previous
import functools

import jax
import jax.numpy as jnp
from jax import lax
from jax.experimental import pallas as pl
from jax.experimental.pallas import tpu as pltpu
from jax.experimental.pallas import tpu_sc as plsc

B, N, CDIM = 8, 2048, 64
S, K = 512, 16
ST, TS = 4, 128
NW = 32
ROWS = B * S * K
R_PER_W = ROWS // NW
CHUNK = 512


def _fps_body(p0_ref, p1_ref, p2_ref, c0_ref, c1_ref, c2_ref):
    p0 = p0_ref[...]
    p1 = p1_ref[...]
    p2 = p2_ref[...]
    lane = lax.broadcasted_iota(jnp.int32, (B, N), 1)
    lane_s = lax.broadcasted_iota(jnp.int32, (B, S), 1)

    def step(i, carry):
        dist, far, a0, a1, a2 = carry
        m = lane == far
        c0 = jnp.sum(jnp.where(m, p0, 0.0), axis=1, keepdims=True)
        c1 = jnp.sum(jnp.where(m, p1, 0.0), axis=1, keepdims=True)
        c2 = jnp.sum(jnp.where(m, p2, 0.0), axis=1, keepdims=True)
        sel = lane_s == i
        a0 = jnp.where(sel, c0, a0)
        a1 = jnp.where(sel, c1, a1)
        a2 = jnp.where(sel, c2, a2)
        d = ((p0 - c0) ** 2 + (p1 - c1) ** 2) + (p2 - c2) ** 2
        dist = jnp.minimum(dist, d)
        mx = jnp.max(dist, axis=1, keepdims=True)
        far = jnp.min(jnp.where(dist == mx, lane, N), axis=1, keepdims=True)
        return dist, far, a0, a1, a2

    dist0 = jnp.full((B, N), 1e10, dtype=jnp.float32)
    far0 = jnp.zeros((B, 1), dtype=jnp.int32)
    z = jnp.zeros((B, S), dtype=jnp.float32)
    init = (dist0, far0, z, z, z)
    _, _, a0, a1, a2 = lax.fori_loop(0, S, step, init)
    c0_ref[...] = a0
    c1_ref[...] = a1
    c2_ref[...] = a2


def _fps(p0, p1, p2, interpret=False):
    return pl.pallas_call(
        _fps_body,
        out_shape=[jax.ShapeDtypeStruct((B, S), jnp.float32)] * 3,
        interpret=interpret,
    )(p0, p1, p2)


def _knn_body(p0_ref, p1_ref, p2_ref, c0_ref, c1_ref, c2_ref, knn_ref):
    lane = lax.broadcasted_iota(jnp.int32, (TS, N), 1)
    kidx = lax.broadcasted_iota(jnp.int32, (TS, K), 1)

    Ds = []
    for b in range(B):
        p0 = p0_ref[b].reshape(1, N)
        p1 = p1_ref[b].reshape(1, N)
        p2 = p2_ref[b].reshape(1, N)
        for t in range(ST):
            c0 = c0_ref[b, t].reshape(TS, 1)
            c1 = c1_ref[b, t].reshape(TS, 1)
            c2 = c2_ref[b, t].reshape(TS, 1)
            Ds.append(((c0 - p0) ** 2 + (c1 - p1) ** 2) + (c2 - p2) ** 2)

    inf = jnp.float32(jnp.inf)
    NT = B * ST

    def step(k, carry):
        dws = list(carry[0])
        iws = list(carry[1])
        accs = list(carry[2])
        for u in range(NT):
            dw, iw = dws[u], iws[u]
            elig = (Ds[u] > dw) | ((Ds[u] == dw) & (lane > iw))
            dv = jnp.where(elig, Ds[u], inf)
            iv = lane
            w = N
            while w > 128:
                h = w // 2
                take = dv[:, h:w] < dv[:, :h]
                dv = jnp.where(take, dv[:, h:w], dv[:, :h])
                iv = jnp.where(take, iv[:, h:w], iv[:, :h])
                w = h
            mn = jnp.min(dv, axis=1, keepdims=True)
            idx = jnp.min(jnp.where(dv == mn, iv, N), axis=1, keepdims=True)
            dws[u] = mn
            iws[u] = idx
            boff = (u // ST) * N
            accs[u] = jnp.where(kidx == k, idx + boff, accs[u])
        return tuple(dws), tuple(iws), tuple(accs)

    acc0 = jnp.zeros((TS, K), jnp.int32)
    neg = jnp.full((TS, 1), -1.0, jnp.float32)
    zi = jnp.zeros((TS, 1), jnp.int32)
    res = lax.fori_loop(
        0, K, step, ((neg,) * NT, (zi,) * NT, (acc0,) * NT))
    for u in range(NT):
        knn_ref[u // ST, u % ST] = res[2][u]


def _knn(p0, p1, p2, c0r, c1r, c2r, interpret=False):
    return pl.pallas_call(
        _knn_body,
        out_shape=jax.ShapeDtypeStruct((B, ST, TS, K), jnp.int32),
        interpret=interpret,
    )(p0.reshape(B, 1, N), p1.reshape(B, 1, N), p2.reshape(B, 1, N),
      c0r, c1r, c2r)


def _make_gather():
    mesh = plsc.VectorSubcoreMesh(
        core_axis_name="c", subcore_axis_name="s", num_cores=2, num_subcores=16
    )

    @functools.partial(
        pl.kernel,
        out_type=jax.ShapeDtypeStruct((ROWS, CDIM), jnp.float32),
        mesh=mesh,
        compiler_params=pltpu.CompilerParams(use_tc_tiling_on_sc=False),
        scratch_types=[
            pltpu.VMEM((CHUNK,), jnp.int32),
            pltpu.VMEM((CHUNK, CDIM), jnp.float32),
            pltpu.SemaphoreType.DMA,
        ],
    )
    def gather_rows(idx_hbm, x_hbm, out_hbm, idx_v, rows_v, sem):
        wid = lax.axis_index("s") * 2 + lax.axis_index("c")
        base = wid * R_PER_W
        for c in range(R_PER_W // CHUNK):
            off = base + c * CHUNK
            pltpu.sync_copy(idx_hbm.at[pl.ds(off, CHUNK)], idx_v)
            pltpu.async_copy(x_hbm.at[idx_v], rows_v, sem).wait()
            pltpu.sync_copy(rows_v, out_hbm.at[pl.ds(off, CHUNK)])

    return gather_rows


_gather_cache = []


def _get_gather():
    if not _gather_cache:
        _gather_cache.append(_make_gather())
    return _gather_cache[0]


def kernel(x):
    p0 = x[:, :, 0]
    p1 = x[:, :, 1]
    p2 = x[:, :, 2]
    c0, c1, c2 = _fps(p0, p1, p2)
    c0r = c0.reshape(B, ST, TS, 1)
    c1r = c1.reshape(B, ST, TS, 1)
    c2r = c2.reshape(B, ST, TS, 1)
    knn = _knn(p0, p1, p2, c0r, c1r, c2r)
    idx_flat = knn.reshape(ROWS)
    rows = _get_gather()(idx_flat, x.reshape(B * N, CDIM))
    return rows.reshape(B, S, K, CDIM)


# speedup vs baseline: 1.1103x; 1.0523x over previous
import functools

import jax
import jax.numpy as jnp
from jax import lax
from jax.experimental import pallas as pl
from jax.experimental.pallas import tpu as pltpu
from jax.experimental.pallas import tpu_sc as plsc

B, N, CDIM = 8, 2048, 64
S, K = 512, 16
ST, TS = 4, 128
NW = 32
ROWS = B * S * K
R_PER_W = ROWS // NW
CHUNK = 512


def _mega_body(p0_ref, p1_ref, p2_ref, knn_ref):
    p0 = p0_ref[...]
    p1 = p1_ref[...]
    p2 = p2_ref[...]
    lane = lax.broadcasted_iota(jnp.int32, (B, N), 1)
    lane_s = lax.broadcasted_iota(jnp.int32, (B, S), 1)
    laneT = lax.broadcasted_iota(jnp.int32, (TS, N), 1)
    kidx = lax.broadcasted_iota(jnp.int32, (TS, K), 1)
    si8 = lax.broadcasted_iota(jnp.int32, (B, 1), 0)
    lT8 = lax.broadcasted_iota(jnp.int32, (TS, B), 1)
    inf = jnp.float32(jnp.inf)

    def fps_step(i, st):
        dist, far, a0, a1, a2 = st
        m = lane == far
        c0 = jnp.sum(jnp.where(m, p0, 0.0), axis=1, keepdims=True)
        c1 = jnp.sum(jnp.where(m, p1, 0.0), axis=1, keepdims=True)
        c2 = jnp.sum(jnp.where(m, p2, 0.0), axis=1, keepdims=True)
        sel = lane_s == i
        a0 = jnp.where(sel, c0, a0)
        a1 = jnp.where(sel, c1, a1)
        a2 = jnp.where(sel, c2, a2)
        d = ((p0 - c0) ** 2 + (p1 - c1) ** 2) + (p2 - c2) ** 2
        dist = jnp.minimum(dist, d)
        mx = jnp.max(dist, axis=1, keepdims=True)
        far = jnp.min(jnp.where(dist == mx, lane, N), axis=1, keepdims=True)
        return dist, far, a0, a1, a2

    dist0 = jnp.full((B, N), 1e10, dtype=jnp.float32)
    far0 = jnp.zeros((B, 1), dtype=jnp.int32)
    z = jnp.zeros((B, S), dtype=jnp.float32)
    st = lax.fori_loop(0, TS, fps_step, (dist0, far0, z, z, z))

    acc0 = jnp.zeros((TS, K), jnp.int32)
    neg1 = jnp.full((TS, 1), -1.0, jnp.float32)
    zi = jnp.zeros((TS, 1), jnp.int32)

    for phase in range(1, 5):
        sec = phase - 1
        a0, a1, a2 = st[2], st[3], st[4]
        sl = slice(sec * TS, (sec + 1) * TS)
        eye = (lax.broadcasted_iota(jnp.int32, (TS, TS), 0)
               == lax.broadcasted_iota(jnp.int32, (TS, TS), 1)
               ).astype(jnp.float32)
        ct0 = jax.lax.dot_general(
            eye, a0[:, sl], (((1,), (1,)), ((), ())),
            preferred_element_type=jnp.float32,
            precision=jax.lax.Precision.HIGHEST)
        ct1 = jax.lax.dot_general(
            eye, a1[:, sl], (((1,), (1,)), ((), ())),
            preferred_element_type=jnp.float32,
            precision=jax.lax.Precision.HIGHEST)
        ct2 = jax.lax.dot_general(
            eye, a2[:, sl], (((1,), (1,)), ((), ())),
            preferred_element_type=jnp.float32,
            precision=jax.lax.Precision.HIGHEST)

        def outer(bb, st, ct0=ct0, ct1=ct1, ct2=ct2, phase=phase):
            mr = si8 == bb
            mc = lT8 == bb
            cs0 = jnp.sum(jnp.where(mc, ct0, 0.0), axis=1, keepdims=True)
            cs1 = jnp.sum(jnp.where(mc, ct1, 0.0), axis=1, keepdims=True)
            cs2 = jnp.sum(jnp.where(mc, ct2, 0.0), axis=1, keepdims=True)
            pr0 = jnp.sum(jnp.where(mr, p0, 0.0), axis=0, keepdims=True)
            pr1 = jnp.sum(jnp.where(mr, p1, 0.0), axis=0, keepdims=True)
            pr2 = jnp.sum(jnp.where(mr, p2, 0.0), axis=0, keepdims=True)
            D = ((cs0 - pr0) ** 2 + (cs1 - pr1) ** 2) + (cs2 - pr2) ** 2
            boff = bb * N

            def inner(j, ist):
                st2, dw, iw, acc = ist
                if phase <= 3:
                    gi = phase * TS + bb * K + j
                    st2 = fps_step(gi, st2)
                elig = (D > dw) | ((D == dw) & (laneT > iw))
                dv = jnp.where(elig, D, inf)
                iv = laneT
                w = N
                while w > 128:
                    h = w // 2
                    take = dv[:, h:w] < dv[:, :h]
                    dv = jnp.where(take, dv[:, h:w], dv[:, :h])
                    iv = jnp.where(take, iv[:, h:w], iv[:, :h])
                    w = h
                mn = jnp.min(dv, axis=1, keepdims=True)
                idx = jnp.min(jnp.where(dv == mn, iv, N),
                              axis=1, keepdims=True)
                acc = jnp.where(kidx == j, idx + boff, acc)
                return st2, mn, idx, acc

            st, _, _, acc = lax.fori_loop(0, K, inner, (st, neg1, zi, acc0))
            knn_ref[bb, sec] = acc
            return st

        for bb in range(B):
            st = outer(bb, st)


def _mega(p0, p1, p2, interpret=False):
    return pl.pallas_call(
        _mega_body,
        out_shape=jax.ShapeDtypeStruct((B, ST, TS, K), jnp.int32),
        interpret=interpret,
    )(p0, p1, p2)


def _make_gather():
    mesh = plsc.VectorSubcoreMesh(
        core_axis_name="c", subcore_axis_name="s", num_cores=2, num_subcores=16
    )

    @functools.partial(
        pl.kernel,
        out_type=jax.ShapeDtypeStruct((ROWS, CDIM), jnp.float32),
        mesh=mesh,
        compiler_params=pltpu.CompilerParams(use_tc_tiling_on_sc=False),
        scratch_types=[
            pltpu.VMEM((CHUNK,), jnp.int32),
            pltpu.VMEM((CHUNK, CDIM), jnp.float32),
            pltpu.SemaphoreType.DMA,
        ],
    )
    def gather_rows(idx_hbm, x_hbm, out_hbm, idx_v, rows_v, sem):
        wid = lax.axis_index("s") * 2 + lax.axis_index("c")
        base = wid * R_PER_W
        for c in range(R_PER_W // CHUNK):
            off = base + c * CHUNK
            pltpu.sync_copy(idx_hbm.at[pl.ds(off, CHUNK)], idx_v)
            pltpu.async_copy(x_hbm.at[idx_v], rows_v, sem).wait()
            pltpu.sync_copy(rows_v, out_hbm.at[pl.ds(off, CHUNK)])

    return gather_rows


_gather_cache = []


def _get_gather():
    if not _gather_cache:
        _gather_cache.append(_make_gather())
    return _gather_cache[0]


def kernel(x):
    p0 = x[:, :, 0]
    p1 = x[:, :, 1]
    p2 = x[:, :, 2]
    knn = _mega(p0, p1, p2)
    idx_flat = knn.reshape(ROWS)
    rows = _get_gather()(idx_flat, x.reshape(B * N, CDIM))
    return rows.reshape(B, S, K, CDIM)


# speedup vs baseline: 1.1663x; 1.0504x over previous
import functools

import jax
import jax.numpy as jnp
from jax import lax
from jax.experimental import pallas as pl
from jax.experimental.pallas import tpu as pltpu
from jax.experimental.pallas import tpu_sc as plsc

B, N, CDIM = 8, 2048, 64
S, K = 512, 16
ST, TS = 4, 128
NW = 32
ROWS = B * S * K
R_PER_W = ROWS // NW
CHUNK = 512


def _mega_body(p0_ref, p1_ref, p2_ref, knn_ref):
    p0 = p0_ref[...]
    p1 = p1_ref[...]
    p2 = p2_ref[...]
    lane = lax.broadcasted_iota(jnp.int32, (B, N), 1)
    lane_s = lax.broadcasted_iota(jnp.int32, (B, S), 1)
    laneT = lax.broadcasted_iota(jnp.int32, (TS, N), 1)
    kidx = lax.broadcasted_iota(jnp.int32, (TS, K), 1)
    si8 = lax.broadcasted_iota(jnp.int32, (B, 1), 0)
    lT8 = lax.broadcasted_iota(jnp.int32, (TS, B), 1)
    inf = jnp.float32(jnp.inf)

    def fps_step(i, st):
        dist, far, a0, a1, a2 = st
        m = lane == far
        c0 = jnp.sum(jnp.where(m, p0, 0.0), axis=1, keepdims=True)
        c1 = jnp.sum(jnp.where(m, p1, 0.0), axis=1, keepdims=True)
        c2 = jnp.sum(jnp.where(m, p2, 0.0), axis=1, keepdims=True)
        sel = lane_s == i
        a0 = jnp.where(sel, c0, a0)
        a1 = jnp.where(sel, c1, a1)
        a2 = jnp.where(sel, c2, a2)
        d = ((p0 - c0) ** 2 + (p1 - c1) ** 2) + (p2 - c2) ** 2
        dist = jnp.minimum(dist, d)
        mx = jnp.max(dist, axis=1, keepdims=True)
        far = jnp.min(jnp.where(dist == mx, lane, N), axis=1, keepdims=True)
        return dist, far, a0, a1, a2

    dist0 = jnp.full((B, N), 1e10, dtype=jnp.float32)
    far0 = jnp.zeros((B, 1), dtype=jnp.int32)
    z = jnp.zeros((B, S), dtype=jnp.float32)
    st = lax.fori_loop(0, TS, fps_step, (dist0, far0, z, z, z))

    acc0 = jnp.zeros((TS, K), jnp.int32)
    neg1 = jnp.full((TS, 1), -1.0, jnp.float32)
    zi = jnp.zeros((TS, 1), jnp.int32)

    for phase in range(1, 4):
        sec = phase - 1
        a0, a1, a2 = st[2], st[3], st[4]
        sl = slice(sec * TS, (sec + 1) * TS)
        eye = (lax.broadcasted_iota(jnp.int32, (TS, TS), 0)
               == lax.broadcasted_iota(jnp.int32, (TS, TS), 1)
               ).astype(jnp.float32)
        ct0 = jax.lax.dot_general(
            eye, a0[:, sl], (((1,), (1,)), ((), ())),
            preferred_element_type=jnp.float32,
            precision=jax.lax.Precision.HIGHEST)
        ct1 = jax.lax.dot_general(
            eye, a1[:, sl], (((1,), (1,)), ((), ())),
            preferred_element_type=jnp.float32,
            precision=jax.lax.Precision.HIGHEST)
        ct2 = jax.lax.dot_general(
            eye, a2[:, sl], (((1,), (1,)), ((), ())),
            preferred_element_type=jnp.float32,
            precision=jax.lax.Precision.HIGHEST)

        def outer(bb, st, ct0=ct0, ct1=ct1, ct2=ct2, phase=phase):
            mr = si8 == bb
            mc = lT8 == bb
            cs0 = jnp.sum(jnp.where(mc, ct0, 0.0), axis=1, keepdims=True)
            cs1 = jnp.sum(jnp.where(mc, ct1, 0.0), axis=1, keepdims=True)
            cs2 = jnp.sum(jnp.where(mc, ct2, 0.0), axis=1, keepdims=True)
            pr0 = jnp.sum(jnp.where(mr, p0, 0.0), axis=0, keepdims=True)
            pr1 = jnp.sum(jnp.where(mr, p1, 0.0), axis=0, keepdims=True)
            pr2 = jnp.sum(jnp.where(mr, p2, 0.0), axis=0, keepdims=True)
            D = ((cs0 - pr0) ** 2 + (cs1 - pr1) ** 2) + (cs2 - pr2) ** 2
            boff = bb * N

            def inner(j, ist):
                st2, dw, iw, acc = ist
                if phase <= 3:
                    gi = phase * TS + bb * K + j
                    st2 = fps_step(gi, st2)
                elig = (D > dw) | ((D == dw) & (laneT > iw))
                dv = jnp.where(elig, D, inf)
                iv = laneT
                w = N
                while w > 128:
                    h = w // 2
                    take = dv[:, h:w] < dv[:, :h]
                    dv = jnp.where(take, dv[:, h:w], dv[:, :h])
                    iv = jnp.where(take, iv[:, h:w], iv[:, :h])
                    w = h
                mn = jnp.min(dv, axis=1, keepdims=True)
                idx = jnp.min(jnp.where(dv == mn, iv, N),
                              axis=1, keepdims=True)
                acc = jnp.where(kidx == j, idx + boff, acc)
                return st2, mn, idx, acc

            st, _, _, acc = lax.fori_loop(0, K, inner, (st, neg1, zi, acc0))
            knn_ref[bb, sec] = acc
            return st

        for bb in range(B):
            st = outer(bb, st)

    sec = 3
    a0, a1, a2 = st[2], st[3], st[4]
    sl = slice(sec * TS, (sec + 1) * TS)
    eye = (lax.broadcasted_iota(jnp.int32, (TS, TS), 0)
           == lax.broadcasted_iota(jnp.int32, (TS, TS), 1)
           ).astype(jnp.float32)
    cts = [jax.lax.dot_general(
        eye, a[:, sl], (((1,), (1,)), ((), ())),
        preferred_element_type=jnp.float32,
        precision=jax.lax.Precision.HIGHEST) for a in (a0, a1, a2)]
    Ds = []
    for bb in range(B):
        mr = si8 == bb
        mc = lT8 == bb
        cs = [jnp.sum(jnp.where(mc, ct, 0.0), axis=1, keepdims=True)
              for ct in cts]
        pr = [jnp.sum(jnp.where(mr, p, 0.0), axis=0, keepdims=True)
              for p in (p0, p1, p2)]
        Ds.append(((cs[0] - pr[0]) ** 2 + (cs[1] - pr[1]) ** 2)
                  + (cs[2] - pr[2]) ** 2)

    def epi(j, ist):
        dws = list(ist[0])
        iws = list(ist[1])
        accs = list(ist[2])
        for bb in range(B):
            elig = ((Ds[bb] > dws[bb])
                    | ((Ds[bb] == dws[bb]) & (laneT > iws[bb])))
            dv = jnp.where(elig, Ds[bb], inf)
            iv = laneT
            w = N
            while w > 128:
                h = w // 2
                take = dv[:, h:w] < dv[:, :h]
                dv = jnp.where(take, dv[:, h:w], dv[:, :h])
                iv = jnp.where(take, iv[:, h:w], iv[:, :h])
                w = h
            mn = jnp.min(dv, axis=1, keepdims=True)
            idx = jnp.min(jnp.where(dv == mn, iv, N), axis=1, keepdims=True)
            dws[bb] = mn
            iws[bb] = idx
            accs[bb] = jnp.where(kidx == j, idx + bb * N, accs[bb])
        return tuple(dws), tuple(iws), tuple(accs)

    res = lax.fori_loop(
        0, K, epi, ((neg1,) * B, (zi,) * B, (acc0,) * B))
    for bb in range(B):
        knn_ref[bb, sec] = res[2][bb]


def _mega(p0, p1, p2, interpret=False):
    return pl.pallas_call(
        _mega_body,
        out_shape=jax.ShapeDtypeStruct((B, ST, TS, K), jnp.int32),
        interpret=interpret,
    )(p0, p1, p2)


def _make_gather():
    mesh = plsc.VectorSubcoreMesh(
        core_axis_name="c", subcore_axis_name="s", num_cores=2, num_subcores=16
    )

    @functools.partial(
        pl.kernel,
        out_type=jax.ShapeDtypeStruct((ROWS, CDIM), jnp.float32),
        mesh=mesh,
        compiler_params=pltpu.CompilerParams(use_tc_tiling_on_sc=False),
        scratch_types=[
            pltpu.VMEM((CHUNK,), jnp.int32),
            pltpu.VMEM((CHUNK, CDIM), jnp.float32),
            pltpu.SemaphoreType.DMA,
        ],
    )
    def gather_rows(idx_hbm, x_hbm, out_hbm, idx_v, rows_v, sem):
        wid = lax.axis_index("s") * 2 + lax.axis_index("c")
        base = wid * R_PER_W
        for c in range(R_PER_W // CHUNK):
            off = base + c * CHUNK
            pltpu.sync_copy(idx_hbm.at[pl.ds(off, CHUNK)], idx_v)
            pltpu.async_copy(x_hbm.at[idx_v], rows_v, sem).wait()
            pltpu.sync_copy(rows_v, out_hbm.at[pl.ds(off, CHUNK)])

    return gather_rows


_gather_cache = []


def _get_gather():
    if not _gather_cache:
        _gather_cache.append(_make_gather())
    return _gather_cache[0]


def kernel(x):
    p0 = x[:, :, 0]
    p1 = x[:, :, 1]
    p2 = x[:, :, 2]
    knn = _mega(p0, p1, p2)
    idx_flat = knn.reshape(ROWS)
    rows = _get_gather()(idx_flat, x.reshape(B * N, CDIM))
    return rows.reshape(B, S, K, CDIM)


# speedup vs baseline: 1.1708x; 1.0038x over previous
import functools

import jax
import jax.numpy as jnp
from jax import lax
from jax.experimental import pallas as pl
from jax.experimental.pallas import tpu as pltpu
from jax.experimental.pallas import tpu_sc as plsc

B, N, CDIM = 8, 2048, 64
S, K = 512, 16
ST, TS = 4, 128
NW = 32
ROWS = B * S * K
R_PER_W = ROWS // NW
CHUNK = 512


def _mega_body(p0_ref, p1_ref, p2_ref, knn_ref):
    p0 = p0_ref[...]
    p1 = p1_ref[...]
    p2 = p2_ref[...]
    lane = lax.broadcasted_iota(jnp.int32, (B, N), 1)
    lane_s = lax.broadcasted_iota(jnp.int32, (B, S), 1)
    laneT = lax.broadcasted_iota(jnp.int32, (TS, N), 1)
    kidx = lax.broadcasted_iota(jnp.int32, (TS, K), 1)
    si8 = lax.broadcasted_iota(jnp.int32, (B, 1), 0)
    lT8 = lax.broadcasted_iota(jnp.int32, (TS, B), 1)
    inf = jnp.float32(jnp.inf)

    def fps_step(i, st):
        dist, far, a0, a1, a2 = st
        m = lane == far
        c0 = jnp.sum(jnp.where(m, p0, 0.0), axis=1, keepdims=True)
        c1 = jnp.sum(jnp.where(m, p1, 0.0), axis=1, keepdims=True)
        c2 = jnp.sum(jnp.where(m, p2, 0.0), axis=1, keepdims=True)
        sel = lane_s == i
        a0 = jnp.where(sel, c0, a0)
        a1 = jnp.where(sel, c1, a1)
        a2 = jnp.where(sel, c2, a2)
        d = ((p0 - c0) ** 2 + (p1 - c1) ** 2) + (p2 - c2) ** 2
        dist = jnp.minimum(dist, d)
        mx = jnp.max(dist, axis=1, keepdims=True)
        far = jnp.min(jnp.where(dist == mx, lane, N), axis=1, keepdims=True)
        return dist, far, a0, a1, a2

    dist0 = jnp.full((B, N), 1e10, dtype=jnp.float32)
    far0 = jnp.zeros((B, 1), dtype=jnp.int32)
    z = jnp.zeros((B, S), dtype=jnp.float32)
    st = lax.fori_loop(0, TS, fps_step, (dist0, far0, z, z, z))

    acc0 = jnp.zeros((TS, K), jnp.int32)
    neg1 = jnp.full((TS, 1), -1.0, jnp.float32)
    zi = jnp.zeros((TS, 1), jnp.int32)

    for phase in range(1, 4):
        sec = phase - 1
        a0, a1, a2 = st[2], st[3], st[4]
        sl = slice(sec * TS, (sec + 1) * TS)
        eye = (lax.broadcasted_iota(jnp.int32, (TS, TS), 0)
               == lax.broadcasted_iota(jnp.int32, (TS, TS), 1)
               ).astype(jnp.float32)
        ct0 = jax.lax.dot_general(
            eye, a0[:, sl], (((1,), (1,)), ((), ())),
            preferred_element_type=jnp.float32,
            precision=jax.lax.Precision.HIGHEST)
        ct1 = jax.lax.dot_general(
            eye, a1[:, sl], (((1,), (1,)), ((), ())),
            preferred_element_type=jnp.float32,
            precision=jax.lax.Precision.HIGHEST)
        ct2 = jax.lax.dot_general(
            eye, a2[:, sl], (((1,), (1,)), ((), ())),
            preferred_element_type=jnp.float32,
            precision=jax.lax.Precision.HIGHEST)

        def outer(bb, st, ct0=ct0, ct1=ct1, ct2=ct2, phase=phase):
            mr = si8 == bb
            mc = lT8 == bb
            cs0 = jnp.sum(jnp.where(mc, ct0, 0.0), axis=1, keepdims=True)
            cs1 = jnp.sum(jnp.where(mc, ct1, 0.0), axis=1, keepdims=True)
            cs2 = jnp.sum(jnp.where(mc, ct2, 0.0), axis=1, keepdims=True)
            pr0 = jnp.sum(jnp.where(mr, p0, 0.0), axis=0, keepdims=True)
            pr1 = jnp.sum(jnp.where(mr, p1, 0.0), axis=0, keepdims=True)
            pr2 = jnp.sum(jnp.where(mr, p2, 0.0), axis=0, keepdims=True)
            D = ((cs0 - pr0) ** 2 + (cs1 - pr1) ** 2) + (cs2 - pr2) ** 2
            boff = bb * N

            def inner(j, ist):
                st2, dw, iw, acc = ist
                if phase <= 3:
                    gi = phase * TS + bb * K + j
                    st2 = fps_step(gi, st2)
                elig = (D > dw) | ((D == dw) & (laneT > iw))
                dv = jnp.where(elig, D, inf)
                iv = laneT
                w = N
                while w > 128:
                    h = w // 2
                    take = dv[:, h:w] < dv[:, :h]
                    dv = jnp.where(take, dv[:, h:w], dv[:, :h])
                    iv = jnp.where(take, iv[:, h:w], iv[:, :h])
                    w = h
                mn = jnp.min(dv, axis=1, keepdims=True)
                idx = jnp.min(jnp.where(dv == mn, iv, N),
                              axis=1, keepdims=True)
                acc = jnp.where(kidx == j, idx + boff, acc)
                return st2, mn, idx, acc

            st, _, _, acc = lax.fori_loop(0, K, inner, (st, neg1, zi, acc0))
            knn_ref[bb, sec] = acc
            return st

        for bb in range(B):
            st = outer(bb, st)

    sec = 3
    a0, a1, a2 = st[2], st[3], st[4]
    sl = slice(sec * TS, (sec + 1) * TS)
    eye = (lax.broadcasted_iota(jnp.int32, (TS, TS), 0)
           == lax.broadcasted_iota(jnp.int32, (TS, TS), 1)
           ).astype(jnp.float32)
    cts = [jax.lax.dot_general(
        eye, a[:, sl], (((1,), (1,)), ((), ())),
        preferred_element_type=jnp.float32,
        precision=jax.lax.Precision.HIGHEST) for a in (a0, a1, a2)]
    Ds = []
    for bb in range(B):
        mr = si8 == bb
        mc = lT8 == bb
        cs = [jnp.sum(jnp.where(mc, ct, 0.0), axis=1, keepdims=True)
              for ct in cts]
        pr = [jnp.sum(jnp.where(mr, p, 0.0), axis=0, keepdims=True)
              for p in (p0, p1, p2)]
        Ds.append(((cs[0] - pr[0]) ** 2 + (cs[1] - pr[1]) ** 2)
                  + (cs[2] - pr[2]) ** 2)

    def epi(j, ist):
        dws = list(ist[0])
        iws = list(ist[1])
        accs = list(ist[2])
        for bb in range(B):
            elig = ((Ds[bb] > dws[bb])
                    | ((Ds[bb] == dws[bb]) & (laneT > iws[bb])))
            dv = jnp.where(elig, Ds[bb], inf)
            iv = laneT
            w = N
            while w > 128:
                h = w // 2
                take = dv[:, h:w] < dv[:, :h]
                dv = jnp.where(take, dv[:, h:w], dv[:, :h])
                iv = jnp.where(take, iv[:, h:w], iv[:, :h])
                w = h
            mn = jnp.min(dv, axis=1, keepdims=True)
            idx = jnp.min(jnp.where(dv == mn, iv, N), axis=1, keepdims=True)
            dws[bb] = mn
            iws[bb] = idx
            accs[bb] = jnp.where(kidx == j, idx + bb * N, accs[bb])
        return tuple(dws), tuple(iws), tuple(accs)

    res = lax.fori_loop(
        0, K, epi, ((neg1,) * B, (zi,) * B, (acc0,) * B))
    for bb in range(B):
        knn_ref[bb, sec] = res[2][bb]


def _mega(p0, p1, p2, interpret=False):
    return pl.pallas_call(
        _mega_body,
        out_shape=jax.ShapeDtypeStruct((B, ST, TS, K), jnp.int32),
        interpret=interpret,
    )(p0, p1, p2)


def _make_gather():
    mesh = plsc.VectorSubcoreMesh(
        core_axis_name="c", subcore_axis_name="s", num_cores=2, num_subcores=16
    )

    @functools.partial(
        pl.kernel,
        out_type=jax.ShapeDtypeStruct((ROWS, CDIM), jnp.float32),
        mesh=mesh,
        compiler_params=pltpu.CompilerParams(use_tc_tiling_on_sc=False),
        scratch_types=[
            pltpu.VMEM((R_PER_W,), jnp.int32),
            pltpu.VMEM((CHUNK, CDIM), jnp.float32),
            pltpu.VMEM((CHUNK, CDIM), jnp.float32),
            pltpu.SemaphoreType.DMA,
            pltpu.SemaphoreType.DMA,
        ],
    )
    def gather_rows(idx_hbm, x_hbm, out_hbm, idx_v, rows0, rows1, s0, s1):
        wid = lax.axis_index("s") * 2 + lax.axis_index("c")
        base = wid * R_PER_W
        pltpu.sync_copy(idx_hbm.at[pl.ds(base, R_PER_W)], idx_v)
        bufs = (rows0, rows1)
        sems = (s0, s1)
        nc = R_PER_W // CHUNK
        pend = []
        for c in range(nc):
            cp = pltpu.async_copy(
                x_hbm.at[idx_v.at[pl.ds(c * CHUNK, CHUNK)]],
                bufs[c % 2], sems[c % 2])
            pend.append(cp)
            if c > 0:
                pend[c - 1].wait()
                pltpu.sync_copy(bufs[(c - 1) % 2],
                                out_hbm.at[pl.ds(base + (c - 1) * CHUNK,
                                                 CHUNK)])
        pend[nc - 1].wait()
        pltpu.sync_copy(bufs[(nc - 1) % 2],
                        out_hbm.at[pl.ds(base + (nc - 1) * CHUNK, CHUNK)])

    return gather_rows


_gather_cache = []


def _get_gather():
    if not _gather_cache:
        _gather_cache.append(_make_gather())
    return _gather_cache[0]


def kernel(x):
    p0 = x[:, :, 0]
    p1 = x[:, :, 1]
    p2 = x[:, :, 2]
    knn = _mega(p0, p1, p2)
    idx_flat = knn.reshape(ROWS)
    rows = _get_gather()(idx_flat, x.reshape(B * N, CDIM))
    return rows.reshape(B, S, K, CDIM)
